# baseline (speedup 1.0000x reference)
import functools

import jax
import jax.numpy as jnp
from jax.experimental import pallas as pl
from jax.experimental.pallas import tpu as pltpu


def _presence_rows(s):
    nc = s.shape[0]
    chans = [s[c] for c in range(nc)]
    m = chans[0]
    for c in range(1, nc):
        m = jnp.maximum(m, chans[c])
    eqs = [chans[c] == m for c in range(nc)]
    flags = []
    seen = None
    for c in range(nc):
        is_lab = eqs[c] if seen is None else jnp.logical_and(eqs[c], jnp.logical_not(seen))
        seen = eqs[c] if seen is None else jnp.logical_or(seen, eqs[c])
        flags.append(jnp.max(is_lab.astype(jnp.float32)))
    return flags


def _fused_body(sa_ref, ia_ref, sb_ref, ib_ref,
                ma_ref, mia_ref, mb_ref, mib_ref, fl_ref, pred_ref):
    b = pl.program_id(0)
    h = pl.program_id(1)

    sa = sa_ref[0]
    sb = sb_ref[0]

    ma_ref[0] = sa
    mb_ref[0] = sb
    wa = sa[0] + sa[1] + sa[2] + sa[3]
    wb = sb[0] + sb[1] + sb[2] + sb[3]
    mia_ref[0] = wa[None, :, :] * ia_ref[0]
    mib_ref[0] = wb[None, :, :] * ib_ref[0]

    fa = _presence_rows(sa)
    fb = _presence_rows(sb)
    vals = fa + fb
    rows = jax.lax.broadcasted_iota(jnp.int32, (8, 128), 0)
    cur = jnp.zeros((8, 128), jnp.float32)
    for i, v in enumerate(vals):
        cur = jnp.where(rows == i, v, cur)

    @pl.when(jnp.logical_and(b == 0, h == 0))
    def _():
        fl_ref[...] = cur

    @pl.when(jnp.logical_not(jnp.logical_and(b == 0, h == 0)))
    def _():
        fl_ref[...] = jnp.maximum(fl_ref[...], cur)

    nb = pl.num_programs(0)
    nh = pl.num_programs(1)

    @pl.when(jnp.logical_and(b == nb - 1, h == nh - 1))
    def _():
        pred_ref[0] = (jnp.min(fl_ref[...]) > 0.5).astype(jnp.int32)


def _fixup_body(cm_ref, sa_ref, ia_ref, sb_ref, ib_ref,
                ma_ref, mia_ref, mb_ref, mib_ref):
    sa = sa_ref[0]
    sb = sb_ref[0]
    nc = sa.shape[0]
    wa = None
    wb = None
    for c in range(nc):
        cmc = cm_ref[0, c]
        mc_a = sa[c] * cmc
        mc_b = sb[c] * cmc
        ma_ref[0, c] = mc_a
        mb_ref[0, c] = mc_b
        wa = mc_a if wa is None else wa + mc_a
        wb = mc_b if wb is None else wb + mc_b
    mia_ref[0] = wa[None, :, :] * ia_ref[0]
    mib_ref[0] = wb[None, :, :] * ib_ref[0]


@functools.partial(jax.jit, static_argnames=("bh", "interpret"))
def _run(img_a, seg_a, img_b, seg_b, bh=256, interpret=False):
    B, C, H, W = img_a.shape
    NC = seg_a.shape[1]
    grid = (B, H // bh)

    seg_spec = pl.BlockSpec((1, NC, bh, W), lambda b, h: (b, 0, h, 0))
    img_spec = pl.BlockSpec((1, C, bh, W), lambda b, h: (b, 0, h, 0))
    fl_spec = pl.BlockSpec((8, 128), lambda b, h: (0, 0))

    f32 = jnp.float32
    ma, mia, mb, mib, flags, pred = pl.pallas_call(
        _fused_body,
        grid=grid,
        in_specs=[seg_spec, img_spec, seg_spec, img_spec],
        out_specs=[seg_spec, img_spec, seg_spec, img_spec, fl_spec,
                   pl.BlockSpec(memory_space=pltpu.SMEM)],
        out_shape=[
            jax.ShapeDtypeStruct((B, NC, H, W), f32),
            jax.ShapeDtypeStruct((B, C, H, W), f32),
            jax.ShapeDtypeStruct((B, NC, H, W), f32),
            jax.ShapeDtypeStruct((B, C, H, W), f32),
            jax.ShapeDtypeStruct((8, 128), f32),
            jax.ShapeDtypeStruct((1,), jnp.int32),
        ],
        interpret=interpret,
    )(seg_a, img_a, seg_b, img_b)

    def fast(_):
        return mia, ma, mib, mb

    def slow(_):
        common = flags[:4, 0] * flags[4:8, 0]
        cm = common.reshape(1, NC)
        cm_spec = pl.BlockSpec(memory_space=pltpu.SMEM)
        o_ma, o_mia, o_mb, o_mib = pl.pallas_call(
            _fixup_body,
            grid=grid,
            in_specs=[cm_spec, seg_spec, img_spec, seg_spec, img_spec],
            out_specs=[seg_spec, img_spec, seg_spec, img_spec],
            out_shape=[
                jax.ShapeDtypeStruct((B, NC, H, W), f32),
                jax.ShapeDtypeStruct((B, C, H, W), f32),
                jax.ShapeDtypeStruct((B, NC, H, W), f32),
                jax.ShapeDtypeStruct((B, C, H, W), f32),
            ],
            interpret=interpret,
        )(cm, seg_a, img_a, seg_b, img_b)
        return o_mia, o_ma, o_mib, o_mb

    return jax.lax.switch(pred[0], [slow, fast], None)


def kernel(img_a, seg_a, img_b, seg_b):
    return _run(img_a, seg_a, img_b, seg_b, bh=256)
